# Initial kernel scaffold; baseline (speedup 1.0000x reference)
import jax
import jax.numpy as jnp
from jax.experimental import pallas as pl


def kernel(x, edge_index, W1, b1, a11w, a11b, a12w, a12b, bias1, W2, b2, a21w, a21b, a22w, a22b, bias2):
    raise NotImplementedError("write your pallas kernel here")


# speedup vs baseline: 17.1772x; 17.1772x over previous
import functools

import jax
import jax.numpy as jnp
from jax import lax
from jax.experimental import pallas as pl
from jax.experimental.pallas import tpu as pltpu
from jax.experimental.pallas import tpu_sc as plsc

N = 10000
NP = 10240
EP = 331776
B = 128
CHUNKS = EP // (32 * B)
PERW = CHUNKS * B
D1 = 144
D2 = 32
BN = 512
GRID = NP // BN


def _k1_body(x_ref, m_ref, brow_ref, out_ref, att2_ref):
    full = (jnp.dot(x_ref[...], m_ref[...],
                    preferred_element_type=jnp.float32) + brow_ref[...])
    d = out_ref.shape[1]
    out_ref[...] = full[:, 0:d]
    att2_ref[...] = full[:, d:d + 16]


def _tc_matmul(x, m, brow, d):
    de = m.shape[1]
    return pl.pallas_call(
        _k1_body,
        grid=(GRID,),
        in_specs=[
            pl.BlockSpec((BN, x.shape[1]), lambda i: (i, 0)),
            pl.BlockSpec((x.shape[1], de), lambda i: (0, 0)),
            pl.BlockSpec((1, de), lambda i: (0, 0)),
        ],
        out_specs=[pl.BlockSpec((BN, d), lambda i: (i, 0)),
                   pl.BlockSpec((BN, 16), lambda i: (i, 0))],
        out_shape=[jax.ShapeDtypeStruct((NP, d), jnp.float32),
                   jax.ShapeDtypeStruct((NP, 16), jnp.float32)],
    )(x, m, brow)


def _k2_body(a_ref, b_ref, e4_ref, b1_ref, m2_ref, b2_ref, out_ref, att2_ref):
    num = a_ref[...] + b_ref[...]
    den4 = jnp.maximum(num[:, 128:132], 1e-30)
    den = jnp.dot(den4, e4_ref[...], preferred_element_type=jnp.float32)
    h = num[:, 0:128] / den + b1_ref[...]
    h = jnp.where(h > 0, h, jnp.exp(jnp.minimum(h, 0.0)) - 1.0)
    full = (jnp.dot(h, m2_ref[...],
                    preferred_element_type=jnp.float32) + b2_ref[...])
    out_ref[...] = full[:, 0:D2]
    att2_ref[...] = full[:, D2:D2 + 16]


def _tc_layer2(acc_a, acc_b, e4, b1row, m2, b2row):
    d2e = D2 + 16
    return pl.pallas_call(
        _k2_body,
        grid=(GRID,),
        in_specs=[
            pl.BlockSpec((BN, D1), lambda i: (i, 0)),
            pl.BlockSpec((BN, D1), lambda i: (i, 0)),
            pl.BlockSpec((4, 128), lambda i: (0, 0)),
            pl.BlockSpec((1, 128), lambda i: (0, 0)),
            pl.BlockSpec((128, d2e), lambda i: (0, 0)),
            pl.BlockSpec((1, d2e), lambda i: (0, 0)),
        ],
        out_specs=[pl.BlockSpec((BN, D2), lambda i: (i, 0)),
                   pl.BlockSpec((BN, 16), lambda i: (i, 0))],
        out_shape=[jax.ShapeDtypeStruct((NP, D2), jnp.float32),
                   jax.ShapeDtypeStruct((NP, 16), jnp.float32)],
    )(acc_a, acc_b, e4, b1row, m2, b2row)


def _k3_body(a_ref, b_ref, b2_ref, out_ref):
    num = a_ref[...] + b_ref[...]
    den = jnp.maximum(num[:, 16:17], 1e-30)
    o = num[:, 0:16] / den + b2_ref[...]
    m = jnp.max(o, axis=1, keepdims=True)
    s = jnp.sum(jnp.exp(o - m), axis=1, keepdims=True)
    out_ref[...] = o - m - jnp.log(s)


def _tc_final(acc_a, acc_b, b2row):
    return pl.pallas_call(
        _k3_body,
        grid=(GRID,),
        in_specs=[
            pl.BlockSpec((BN, D2), lambda i: (i, 0)),
            pl.BlockSpec((BN, D2), lambda i: (i, 0)),
            pl.BlockSpec((1, 16), lambda i: (0, 0)),
        ],
        out_specs=pl.BlockSpec((BN, 16), lambda i: (i, 0)),
        out_shape=jax.ShapeDtypeStruct((NP, 16), jnp.float32),
    )(acc_a, acc_b, b2row)


def _sc_edge_kernel(d, att_cols, nheads, head_w):
    rows_per_tile = NP // 16

    @functools.partial(
        pl.kernel,
        out_type=jax.ShapeDtypeStruct((2, NP, d), jnp.float32),
        mesh=plsc.VectorSubcoreMesh(core_axis_name="c", subcore_axis_name="s"),
        compiler_params=pltpu.CompilerParams(needs_layout_passes=False,
                                             use_tc_tiling_on_sc=False),
        scratch_types=[
            pltpu.VMEM_SHARED((NP, d), jnp.float32),
            pltpu.VMEM_SHARED((NP, 16), jnp.float32),
            pltpu.VMEM((2, B), jnp.int32),
            pltpu.VMEM((B, d), jnp.float32),
            pltpu.VMEM((B, 16), jnp.float32),
        ],
    )
    def k(txext, srcdst, att2t, out, acc, att2_sh, idx_v, rows_v, att2g_v):
        cid = lax.axis_index("c")
        sid = lax.axis_index("s")
        wid = sid * 2 + cid
        iota = lax.iota(jnp.int32, 16)
        zv = jnp.zeros((16,), jnp.float32)

        for r in range(B):
            for c in range(0, d, 16):
                rows_v[r, pl.ds(c, 16)] = zv
        r0 = sid * rows_per_tile
        for t in range(rows_per_tile // B):
            pltpu.sync_copy(rows_v, acc.at[pl.ds(r0 + t * B, B)])
        pltpu.sync_copy(att2t.at[pl.ds(r0, rows_per_tile)],
                        att2_sh.at[pl.ds(r0, rows_per_tile)])
        plsc.subcore_barrier()

        def chunk(ci, _):
            base = wid * PERW + ci * B
            pltpu.sync_copy(srcdst.at[0, pl.ds(base, B)], idx_v.at[0])
            pltpu.sync_copy(srcdst.at[1, pl.ds(base, B)], idx_v.at[1])
            pltpu.sync_copy(txext.at[idx_v.at[0]], rows_v)
            pltpu.sync_copy(att2_sh.at[idx_v.at[1]], att2g_v)

            for g in range(B // 16):
                rvec = iota + g * 16
                for h in range(nheads):
                    csp = jnp.full((16,), att_cols + h, jnp.int32)
                    a1 = plsc.load_gather(rows_v, [rvec, csp])
                    a2 = plsc.load_gather(att2g_v,
                                          [rvec, jnp.full((16,), h, jnp.int32)])
                    e = a1 + a2
                    e = jnp.minimum(jnp.maximum(e, 0.2 * e), 60.0)
                    plsc.store_scatter(rows_v, [rvec, csp], jnp.exp(e))

            def scale(j, _):
                jb = jnp.full((16,), j, jnp.int32)
                for h in range(nheads):
                    wsp = plsc.load_gather(
                        rows_v, [jb, jnp.full((16,), att_cols + h, jnp.int32)])
                    for kk in range(head_w // 16):
                        sl = pl.ds(h * head_w + kk * 16, 16)
                        rows_v[j, sl] = rows_v[j, sl] * wsp
                return 0

            lax.fori_loop(0, B, scale, 0)

            pltpu.sync_copy(rows_v, acc.at[idx_v.at[1]], add=True)
            return 0

        lax.fori_loop(0, CHUNKS, chunk, 0)
        plsc.subcore_barrier()
        pltpu.sync_copy(acc.at[pl.ds(sid * rows_per_tile, rows_per_tile)],
                        out.at[cid, pl.ds(sid * rows_per_tile, rows_per_tile)])

    return k


@functools.lru_cache(maxsize=None)
def _get_sc_kernels():
    return _sc_edge_kernel(D1, 128, 4, 32), _sc_edge_kernel(D2, 16, 1, 16)


def kernel(x, edge_index, W1, b1, a11w, a11b, a12w, a12b, bias1,
           W2, b2, a21w, a21b, a22w, a22b, bias2):
    H, O, Din = W1.shape
    C = W2.shape[1]
    Hid = H * O

    att2w1 = jnp.einsum('ho,hod->dh', a12w, W1)
    att2b1 = jnp.sum(b1 * a12w, axis=1) + a12b
    m1 = jnp.zeros((Din, D1 + 16), jnp.float32)
    m1 = m1.at[:, 0:Hid].set(W1.transpose(2, 0, 1).reshape(Din, Hid))
    m1 = m1.at[:, Hid:Hid + H].set(jnp.einsum('ho,hod->dh', a11w, W1))
    m1 = m1.at[:, Hid + H:Hid + 2 * H].set(att2w1)
    m1 = m1.at[:, D1:D1 + H].set(att2w1)
    brow1 = jnp.zeros((1, D1 + 16), jnp.float32)
    brow1 = brow1.at[0, 0:Hid].set(b1.reshape(Hid))
    brow1 = brow1.at[0, Hid:Hid + H].set(jnp.sum(b1 * a11w, axis=1) + a11b)
    brow1 = brow1.at[0, Hid + H:Hid + 2 * H].set(att2b1)
    brow1 = brow1.at[0, D1:D1 + H].set(att2b1)

    att2w2 = W2[0].T @ a22w[0]
    att2b2 = jnp.dot(b2[0], a22w[0]) + a22b[0]
    m2 = jnp.zeros((Hid, D2 + 16), jnp.float32)
    m2 = m2.at[:, 0:C].set(W2[0].T)
    m2 = m2.at[:, C].set(W2[0].T @ a21w[0])
    m2 = m2.at[:, C + 1].set(att2w2)
    m2 = m2.at[:, D2].set(att2w2)
    brow2 = jnp.zeros((1, D2 + 16), jnp.float32)
    brow2 = brow2.at[0, 0:C].set(b2[0])
    brow2 = brow2.at[0, C].set(jnp.dot(b2[0], a21w[0]) + a21b[0])
    brow2 = brow2.at[0, C + 1].set(att2b2)
    brow2 = brow2.at[0, D2].set(att2b2)

    e4 = jnp.zeros((4, 128), jnp.float32)
    for h in range(4):
        e4 = e4.at[h, h * 32:(h + 1) * 32].set(1.0)

    xp = jnp.zeros((NP, Din), jnp.float32).at[0:N].set(x)
    sl = jnp.arange(N, dtype=jnp.int32)
    pad = jnp.full((EP - edge_index.shape[1] - N,), N, jnp.int32)
    src = jnp.concatenate([edge_index[0].astype(jnp.int32), sl, pad])
    dst = jnp.concatenate([edge_index[1].astype(jnp.int32), sl, pad])
    srcdst = jnp.stack([src, dst])

    sc1, sc2 = _get_sc_kernels()

    tx_ext, att2t1 = _tc_matmul(xp, m1, brow1, D1)
    part1 = sc1(tx_ext, srcdst, att2t1)

    tx2_ext, att2t2 = _tc_layer2(part1[0], part1[1], e4,
                                 bias1[None, :].astype(jnp.float32), m2, brow2)
    part2 = sc2(tx2_ext, srcdst, att2t2)

    out = _tc_final(part2[0], part2[1], bias2[None, :].astype(jnp.float32))
    return out[0:N]


# speedup vs baseline: 23.4271x; 1.3638x over previous
import functools

import jax
import jax.numpy as jnp
from jax import lax
from jax.experimental import pallas as pl
from jax.experimental.pallas import tpu as pltpu
from jax.experimental.pallas import tpu_sc as plsc

N = 10000
NP = 10240
EP = 331776
B = 64
CHUNKS = EP // (32 * B)
SUPER = 18
PERW = CHUNKS * B
D1 = 144
D2 = 32
BN = 512
GRID = NP // BN


def _k1_body(x_ref, m_ref, brow_ref, out_ref, att2_ref):
    full = (jnp.dot(x_ref[...], m_ref[...],
                    preferred_element_type=jnp.float32) + brow_ref[...])
    d = out_ref.shape[1]
    out_ref[...] = full[:, 0:d]
    att2_ref[...] = full[:, d:d + 16]


def _tc_matmul(x, m, brow, d):
    de = m.shape[1]
    return pl.pallas_call(
        _k1_body,
        grid=(GRID,),
        in_specs=[
            pl.BlockSpec((BN, x.shape[1]), lambda i: (i, 0)),
            pl.BlockSpec((x.shape[1], de), lambda i: (0, 0)),
            pl.BlockSpec((1, de), lambda i: (0, 0)),
        ],
        out_specs=[pl.BlockSpec((BN, d), lambda i: (i, 0)),
                   pl.BlockSpec((BN, 16), lambda i: (i, 0))],
        out_shape=[jax.ShapeDtypeStruct((NP, d), jnp.float32),
                   jax.ShapeDtypeStruct((NP, 16), jnp.float32)],
    )(x, m, brow)


def _k2_body(a_ref, b_ref, e4_ref, b1_ref, m2_ref, b2_ref, out_ref, att2_ref):
    num = a_ref[...] + b_ref[...]
    den4 = jnp.maximum(num[:, 128:132], 1e-30)
    den = jnp.dot(den4, e4_ref[...], preferred_element_type=jnp.float32)
    h = num[:, 0:128] / den + b1_ref[...]
    h = jnp.where(h > 0, h, jnp.exp(jnp.minimum(h, 0.0)) - 1.0)
    full = (jnp.dot(h, m2_ref[...],
                    preferred_element_type=jnp.float32) + b2_ref[...])
    out_ref[...] = full[:, 0:D2]
    att2_ref[...] = full[:, D2:D2 + 16]


def _tc_layer2(acc_a, acc_b, e4, b1row, m2, b2row):
    d2e = D2 + 16
    return pl.pallas_call(
        _k2_body,
        grid=(GRID,),
        in_specs=[
            pl.BlockSpec((BN, D1), lambda i: (i, 0)),
            pl.BlockSpec((BN, D1), lambda i: (i, 0)),
            pl.BlockSpec((4, 128), lambda i: (0, 0)),
            pl.BlockSpec((1, 128), lambda i: (0, 0)),
            pl.BlockSpec((128, d2e), lambda i: (0, 0)),
            pl.BlockSpec((1, d2e), lambda i: (0, 0)),
        ],
        out_specs=[pl.BlockSpec((BN, D2), lambda i: (i, 0)),
                   pl.BlockSpec((BN, 16), lambda i: (i, 0))],
        out_shape=[jax.ShapeDtypeStruct((NP, D2), jnp.float32),
                   jax.ShapeDtypeStruct((NP, 16), jnp.float32)],
    )(acc_a, acc_b, e4, b1row, m2, b2row)


def _k3_body(a_ref, b_ref, b2_ref, out_ref):
    num = a_ref[...] + b_ref[...]
    den = jnp.maximum(num[:, 16:17], 1e-30)
    o = num[:, 0:16] / den + b2_ref[...]
    m = jnp.max(o, axis=1, keepdims=True)
    s = jnp.sum(jnp.exp(o - m), axis=1, keepdims=True)
    out_ref[...] = o - m - jnp.log(s)


def _tc_final(acc_a, acc_b, b2row):
    return pl.pallas_call(
        _k3_body,
        grid=(GRID,),
        in_specs=[
            pl.BlockSpec((BN, D2), lambda i: (i, 0)),
            pl.BlockSpec((BN, D2), lambda i: (i, 0)),
            pl.BlockSpec((1, 16), lambda i: (0, 0)),
        ],
        out_specs=pl.BlockSpec((BN, 16), lambda i: (i, 0)),
        out_shape=jax.ShapeDtypeStruct((NP, 16), jnp.float32),
    )(acc_a, acc_b, b2row)


def _sc_edge_kernel(d, att_cols, nheads, head_w):
    rows_per_tile = NP // 16

    @functools.partial(
        pl.kernel,
        out_type=jax.ShapeDtypeStruct((2, NP, d), jnp.float32),
        mesh=plsc.VectorSubcoreMesh(core_axis_name="c", subcore_axis_name="s"),
        compiler_params=pltpu.CompilerParams(needs_layout_passes=False,
                                             use_tc_tiling_on_sc=False),
        scratch_types=[
            pltpu.VMEM_SHARED((NP, d), jnp.float32),
            pltpu.VMEM_SHARED((NP, 16), jnp.float32),
            pltpu.VMEM((SUPER, B), jnp.int32),
            pltpu.VMEM((SUPER, B), jnp.int32),
            pltpu.VMEM((B, d), jnp.float32),
            pltpu.VMEM((B, d), jnp.float32),
            pltpu.VMEM((B, 16), jnp.float32),
            pltpu.VMEM((B, 16), jnp.float32),
            pltpu.SemaphoreType.DMA,
            pltpu.SemaphoreType.DMA,
        ],
    )
    def k(txext, src3, dst3, att2t, out, acc, att2_sh, isrc_v, idst_v,
          rows0_v, rows1_v, att2g0_v, att2g1_v, sem0, sem1):
        cid = lax.axis_index("c")
        sid = lax.axis_index("s")
        wid = sid * 2 + cid
        iota = lax.iota(jnp.int32, 16)
        zv = jnp.zeros((16,), jnp.float32)
        rows_b = (rows0_v, rows1_v)
        att2g_b = (att2g0_v, att2g1_v)
        sem_b = (sem0, sem1)

        for r in range(B):
            for c in range(0, d, 16):
                rows0_v[r, pl.ds(c, 16)] = zv
        r0 = sid * rows_per_tile
        for t in range(rows_per_tile // B):
            pltpu.sync_copy(rows0_v, acc.at[pl.ds(r0 + t * B, B)])
        pltpu.sync_copy(att2t.at[pl.ds(r0, rows_per_tile)],
                        att2_sh.at[pl.ds(r0, rows_per_tile)])
        plsc.subcore_barrier()

        wrow = wid * CHUNKS

        def fire(c, b):
            return pltpu.async_copy(txext.at[isrc_v.at[c]], rows_b[b],
                                    sem_b[b])

        def block(bi, _):
            base = wrow + bi * SUPER
            pltpu.sync_copy(src3.at[pl.ds(base, SUPER)], isrc_v)
            pltpu.sync_copy(dst3.at[pl.ds(base, SUPER)], idst_v)
            pend = fire(0, 0)
            for c in range(SUPER):
                b = c % 2
                rows_v = rows_b[b]
                att2g_v = att2g_b[b]
                cur = pend
                if c + 1 < SUPER:
                    pend = fire(c + 1, 1 - b)
                cur.wait()
                pltpu.sync_copy(att2_sh.at[idst_v.at[c]], att2g_v)

                for g in range(B // 16):
                    rvec = iota + g * 16
                    for h in range(nheads):
                        csp = jnp.full((16,), att_cols + h, jnp.int32)
                        a1 = plsc.load_gather(rows_v, [rvec, csp])
                        a2 = plsc.load_gather(
                            att2g_v, [rvec, jnp.full((16,), h, jnp.int32)])
                        e = a1 + a2
                        e = jnp.minimum(jnp.maximum(e, 0.2 * e), 60.0)
                        plsc.store_scatter(rows_v, [rvec, csp], jnp.exp(e))

                def scale(j, _):
                    jb = jnp.full((16,), j, jnp.int32)
                    for h in range(nheads):
                        wsp = plsc.load_gather(
                            rows_v,
                            [jb, jnp.full((16,), att_cols + h, jnp.int32)])
                        for kk in range(head_w // 16):
                            sl = pl.ds(h * head_w + kk * 16, 16)
                            rows_v[j, sl] = rows_v[j, sl] * wsp
                    return 0

                lax.fori_loop(0, B, scale, 0)

                pltpu.sync_copy(rows_v, acc.at[idst_v.at[c]], add=True)
            return 0

        lax.fori_loop(0, CHUNKS // SUPER, block, 0)
        plsc.subcore_barrier()
        pltpu.sync_copy(acc.at[pl.ds(sid * rows_per_tile, rows_per_tile)],
                        out.at[cid, pl.ds(sid * rows_per_tile, rows_per_tile)])

    return k


@functools.lru_cache(maxsize=None)
def _get_sc_kernels():
    return _sc_edge_kernel(D1, 128, 4, 32), _sc_edge_kernel(D2, 16, 1, 16)


def kernel(x, edge_index, W1, b1, a11w, a11b, a12w, a12b, bias1,
           W2, b2, a21w, a21b, a22w, a22b, bias2):
    H, O, Din = W1.shape
    C = W2.shape[1]
    Hid = H * O

    att2w1 = jnp.einsum('ho,hod->dh', a12w, W1)
    att2b1 = jnp.sum(b1 * a12w, axis=1) + a12b
    m1 = jnp.zeros((Din, D1 + 16), jnp.float32)
    m1 = m1.at[:, 0:Hid].set(W1.transpose(2, 0, 1).reshape(Din, Hid))
    m1 = m1.at[:, Hid:Hid + H].set(jnp.einsum('ho,hod->dh', a11w, W1))
    m1 = m1.at[:, Hid + H:Hid + 2 * H].set(att2w1)
    m1 = m1.at[:, D1:D1 + H].set(att2w1)
    brow1 = jnp.zeros((1, D1 + 16), jnp.float32)
    brow1 = brow1.at[0, 0:Hid].set(b1.reshape(Hid))
    brow1 = brow1.at[0, Hid:Hid + H].set(jnp.sum(b1 * a11w, axis=1) + a11b)
    brow1 = brow1.at[0, Hid + H:Hid + 2 * H].set(att2b1)
    brow1 = brow1.at[0, D1:D1 + H].set(att2b1)

    att2w2 = W2[0].T @ a22w[0]
    att2b2 = jnp.dot(b2[0], a22w[0]) + a22b[0]
    m2 = jnp.zeros((Hid, D2 + 16), jnp.float32)
    m2 = m2.at[:, 0:C].set(W2[0].T)
    m2 = m2.at[:, C].set(W2[0].T @ a21w[0])
    m2 = m2.at[:, C + 1].set(att2w2)
    m2 = m2.at[:, D2].set(att2w2)
    brow2 = jnp.zeros((1, D2 + 16), jnp.float32)
    brow2 = brow2.at[0, 0:C].set(b2[0])
    brow2 = brow2.at[0, C].set(jnp.dot(b2[0], a21w[0]) + a21b[0])
    brow2 = brow2.at[0, C + 1].set(att2b2)
    brow2 = brow2.at[0, D2].set(att2b2)

    e4 = jnp.zeros((4, 128), jnp.float32)
    for h in range(4):
        e4 = e4.at[h, h * 32:(h + 1) * 32].set(1.0)

    xp = jnp.zeros((NP, Din), jnp.float32).at[0:N].set(x)
    sl = jnp.arange(N, dtype=jnp.int32)
    pad = jnp.full((EP - edge_index.shape[1] - N,), N, jnp.int32)
    src3 = jnp.concatenate([edge_index[0].astype(jnp.int32), sl, pad]).reshape(-1, B)
    dst3 = jnp.concatenate([edge_index[1].astype(jnp.int32), sl, pad]).reshape(-1, B)

    sc1, sc2 = _get_sc_kernels()

    tx_ext, att2t1 = _tc_matmul(xp, m1, brow1, D1)
    part1 = sc1(tx_ext, src3, dst3, att2t1)

    tx2_ext, att2t2 = _tc_layer2(part1[0], part1[1], e4,
                                 bias1[None, :].astype(jnp.float32), m2, brow2)
    part2 = sc2(tx2_ext, src3, dst3, att2t2)

    out = _tc_final(part2[0], part2[1], bias2[None, :].astype(jnp.float32))
    return out[0:N]


# speedup vs baseline: 23.4342x; 1.0003x over previous
import functools

import jax
import jax.numpy as jnp
from jax import lax
from jax.experimental import pallas as pl
from jax.experimental.pallas import tpu as pltpu
from jax.experimental.pallas import tpu_sc as plsc

N = 10000
NP = 10240
EP = 331776
B = 64
CHUNKS = EP // (32 * B)
SUPER = 18
PERW = CHUNKS * B
D1 = 144
D2 = 32
BN = 512
GRID = NP // BN


def _k1_body(x_ref, m_ref, brow_ref, out_ref, att2_ref):
    full = (jnp.dot(x_ref[...], m_ref[...],
                    preferred_element_type=jnp.float32) + brow_ref[...])
    d = out_ref.shape[1]
    out_ref[...] = full[:, 0:d]
    att2_ref[...] = full[:, d:d + 16]


def _tc_matmul(x, m, brow, d):
    de = m.shape[1]
    return pl.pallas_call(
        _k1_body,
        grid=(GRID,),
        in_specs=[
            pl.BlockSpec((BN, x.shape[1]), lambda i: (i, 0)),
            pl.BlockSpec((x.shape[1], de), lambda i: (0, 0)),
            pl.BlockSpec((1, de), lambda i: (0, 0)),
        ],
        out_specs=[pl.BlockSpec((BN, d), lambda i: (i, 0)),
                   pl.BlockSpec((BN, 16), lambda i: (i, 0))],
        out_shape=[jax.ShapeDtypeStruct((NP, d), jnp.float32),
                   jax.ShapeDtypeStruct((NP, 16), jnp.float32)],
    )(x, m, brow)


def _k2_body(a_ref, b_ref, e4_ref, b1_ref, m2_ref, b2_ref, out_ref, att2_ref):
    num = a_ref[...] + b_ref[...]
    den4 = jnp.maximum(num[:, 128:132], 1e-30)
    den = jnp.dot(den4, e4_ref[...], preferred_element_type=jnp.float32)
    h = num[:, 0:128] / den + b1_ref[...]
    h = jnp.where(h > 0, h, jnp.exp(jnp.minimum(h, 0.0)) - 1.0)
    full = (jnp.dot(h, m2_ref[...],
                    preferred_element_type=jnp.float32) + b2_ref[...])
    out_ref[...] = full[:, 0:D2]
    att2_ref[...] = full[:, D2:D2 + 16]


def _tc_layer2(acc_a, acc_b, e4, b1row, m2, b2row):
    d2e = D2 + 16
    return pl.pallas_call(
        _k2_body,
        grid=(GRID,),
        in_specs=[
            pl.BlockSpec((BN, D1), lambda i: (i, 0)),
            pl.BlockSpec((BN, D1), lambda i: (i, 0)),
            pl.BlockSpec((4, 128), lambda i: (0, 0)),
            pl.BlockSpec((1, 128), lambda i: (0, 0)),
            pl.BlockSpec((128, d2e), lambda i: (0, 0)),
            pl.BlockSpec((1, d2e), lambda i: (0, 0)),
        ],
        out_specs=[pl.BlockSpec((BN, D2), lambda i: (i, 0)),
                   pl.BlockSpec((BN, 16), lambda i: (i, 0))],
        out_shape=[jax.ShapeDtypeStruct((NP, D2), jnp.float32),
                   jax.ShapeDtypeStruct((NP, 16), jnp.float32)],
    )(acc_a, acc_b, e4, b1row, m2, b2row)


def _k3_body(a_ref, b_ref, b2_ref, out_ref):
    num = a_ref[...] + b_ref[...]
    den = jnp.maximum(num[:, 16:17], 1e-30)
    o = num[:, 0:16] / den + b2_ref[...]
    m = jnp.max(o, axis=1, keepdims=True)
    s = jnp.sum(jnp.exp(o - m), axis=1, keepdims=True)
    out_ref[...] = o - m - jnp.log(s)


def _tc_final(acc_a, acc_b, b2row):
    return pl.pallas_call(
        _k3_body,
        grid=(GRID,),
        in_specs=[
            pl.BlockSpec((BN, D2), lambda i: (i, 0)),
            pl.BlockSpec((BN, D2), lambda i: (i, 0)),
            pl.BlockSpec((1, 16), lambda i: (0, 0)),
        ],
        out_specs=pl.BlockSpec((BN, 16), lambda i: (i, 0)),
        out_shape=jax.ShapeDtypeStruct((NP, 16), jnp.float32),
    )(acc_a, acc_b, b2row)


def _sc_edge_kernel(d, att_cols, nheads, head_w):
    rows_per_tile = NP // 16

    @functools.partial(
        pl.kernel,
        out_type=jax.ShapeDtypeStruct((2, NP, d), jnp.float32),
        mesh=plsc.VectorSubcoreMesh(core_axis_name="c", subcore_axis_name="s"),
        compiler_params=pltpu.CompilerParams(needs_layout_passes=False,
                                             use_tc_tiling_on_sc=False),
        scratch_types=[
            pltpu.VMEM_SHARED((NP, d), jnp.float32),
            pltpu.VMEM_SHARED((NP, 16), jnp.float32),
            pltpu.VMEM((SUPER, B), jnp.int32),
            pltpu.VMEM((SUPER, B), jnp.int32),
            pltpu.VMEM((B, d), jnp.float32),
            pltpu.VMEM((B, d), jnp.float32),
            pltpu.VMEM((B, 16), jnp.float32),
            pltpu.VMEM((B, 16), jnp.float32),
            pltpu.SemaphoreType.DMA,
            pltpu.SemaphoreType.DMA,
            pltpu.SemaphoreType.DMA,
            pltpu.SemaphoreType.DMA,
        ],
    )
    def k(txext, src3, dst3, att2t, out, acc, att2_sh, isrc_v, idst_v,
          rows0_v, rows1_v, att2g0_v, att2g1_v, sem0, sem1, ssem0, ssem1):
        cid = lax.axis_index("c")
        sid = lax.axis_index("s")
        wid = sid * 2 + cid
        iota = lax.iota(jnp.int32, 16)
        zv = jnp.zeros((16,), jnp.float32)
        rows_b = (rows0_v, rows1_v)
        att2g_b = (att2g0_v, att2g1_v)
        sem_b = (sem0, sem1)
        ssem_b = (ssem0, ssem1)

        for r in range(B):
            for c in range(0, d, 16):
                rows0_v[r, pl.ds(c, 16)] = zv
        r0 = sid * rows_per_tile
        for t in range(rows_per_tile // B):
            pltpu.sync_copy(rows0_v, acc.at[pl.ds(r0 + t * B, B)])
        pltpu.sync_copy(att2t.at[pl.ds(r0, rows_per_tile)],
                        att2_sh.at[pl.ds(r0, rows_per_tile)])
        plsc.subcore_barrier()

        wrow = wid * CHUNKS

        def fire(c, b):
            return pltpu.async_copy(txext.at[isrc_v.at[c]], rows_b[b],
                                    sem_b[b])

        def block(bi, _):
            base = wrow + bi * SUPER
            pltpu.sync_copy(src3.at[pl.ds(base, SUPER)], isrc_v)
            pltpu.sync_copy(dst3.at[pl.ds(base, SUPER)], idst_v)
            pend = fire(0, 0)
            spend = [None, None]
            for c in range(SUPER):
                b = c % 2
                rows_v = rows_b[b]
                att2g_v = att2g_b[b]
                cur = pend
                if c + 1 < SUPER:
                    if spend[1 - b] is not None:
                        spend[1 - b].wait()
                        spend[1 - b] = None
                    pend = fire(c + 1, 1 - b)
                cur.wait()
                pltpu.sync_copy(att2_sh.at[idst_v.at[c]], att2g_v)

                for g in range(B // 16):
                    rvec = iota + g * 16
                    for h in range(nheads):
                        csp = jnp.full((16,), att_cols + h, jnp.int32)
                        a1 = plsc.load_gather(rows_v, [rvec, csp])
                        a2 = plsc.load_gather(
                            att2g_v, [rvec, jnp.full((16,), h, jnp.int32)])
                        e = a1 + a2
                        e = jnp.minimum(jnp.maximum(e, 0.2 * e), 60.0)
                        plsc.store_scatter(rows_v, [rvec, csp], jnp.exp(e))

                def scale(j2, _):
                    for dj in range(2):
                        j = j2 * 2 + dj
                        jb = jnp.full((16,), j, jnp.int32)
                        for h in range(nheads):
                            wsp = plsc.load_gather(
                                rows_v,
                                [jb, jnp.full((16,), att_cols + h, jnp.int32)])
                            for kk in range(head_w // 16):
                                sl = pl.ds(h * head_w + kk * 16, 16)
                                rows_v[j, sl] = rows_v[j, sl] * wsp
                    return 0

                lax.fori_loop(0, B // 2, scale, 0)

                spend[b] = pltpu.async_copy(rows_v, acc.at[idst_v.at[c]],
                                            ssem_b[b], add=True)
            for b in range(2):
                if spend[b] is not None:
                    spend[b].wait()
            return 0

        lax.fori_loop(0, CHUNKS // SUPER, block, 0)
        plsc.subcore_barrier()
        pltpu.sync_copy(acc.at[pl.ds(sid * rows_per_tile, rows_per_tile)],
                        out.at[cid, pl.ds(sid * rows_per_tile, rows_per_tile)])

    return k


@functools.lru_cache(maxsize=None)
def _get_sc_kernels():
    return _sc_edge_kernel(D1, 128, 4, 32), _sc_edge_kernel(D2, 16, 1, 16)


def kernel(x, edge_index, W1, b1, a11w, a11b, a12w, a12b, bias1,
           W2, b2, a21w, a21b, a22w, a22b, bias2):
    H, O, Din = W1.shape
    C = W2.shape[1]
    Hid = H * O

    att2w1 = jnp.einsum('ho,hod->dh', a12w, W1)
    att2b1 = jnp.sum(b1 * a12w, axis=1) + a12b
    m1 = jnp.zeros((Din, D1 + 16), jnp.float32)
    m1 = m1.at[:, 0:Hid].set(W1.transpose(2, 0, 1).reshape(Din, Hid))
    m1 = m1.at[:, Hid:Hid + H].set(jnp.einsum('ho,hod->dh', a11w, W1))
    m1 = m1.at[:, Hid + H:Hid + 2 * H].set(att2w1)
    m1 = m1.at[:, D1:D1 + H].set(att2w1)
    brow1 = jnp.zeros((1, D1 + 16), jnp.float32)
    brow1 = brow1.at[0, 0:Hid].set(b1.reshape(Hid))
    brow1 = brow1.at[0, Hid:Hid + H].set(jnp.sum(b1 * a11w, axis=1) + a11b)
    brow1 = brow1.at[0, Hid + H:Hid + 2 * H].set(att2b1)
    brow1 = brow1.at[0, D1:D1 + H].set(att2b1)

    att2w2 = W2[0].T @ a22w[0]
    att2b2 = jnp.dot(b2[0], a22w[0]) + a22b[0]
    m2 = jnp.zeros((Hid, D2 + 16), jnp.float32)
    m2 = m2.at[:, 0:C].set(W2[0].T)
    m2 = m2.at[:, C].set(W2[0].T @ a21w[0])
    m2 = m2.at[:, C + 1].set(att2w2)
    m2 = m2.at[:, D2].set(att2w2)
    brow2 = jnp.zeros((1, D2 + 16), jnp.float32)
    brow2 = brow2.at[0, 0:C].set(b2[0])
    brow2 = brow2.at[0, C].set(jnp.dot(b2[0], a21w[0]) + a21b[0])
    brow2 = brow2.at[0, C + 1].set(att2b2)
    brow2 = brow2.at[0, D2].set(att2b2)

    e4 = jnp.zeros((4, 128), jnp.float32)
    for h in range(4):
        e4 = e4.at[h, h * 32:(h + 1) * 32].set(1.0)

    xp = jnp.zeros((NP, Din), jnp.float32).at[0:N].set(x)
    sl = jnp.arange(N, dtype=jnp.int32)
    pad = jnp.full((EP - edge_index.shape[1] - N,), N, jnp.int32)
    src3 = jnp.concatenate([edge_index[0].astype(jnp.int32), sl, pad]).reshape(-1, B)
    dst3 = jnp.concatenate([edge_index[1].astype(jnp.int32), sl, pad]).reshape(-1, B)

    sc1, sc2 = _get_sc_kernels()

    tx_ext, att2t1 = _tc_matmul(xp, m1, brow1, D1)
    part1 = sc1(tx_ext, src3, dst3, att2t1)

    tx2_ext, att2t2 = _tc_layer2(part1[0], part1[1], e4,
                                 bias1[None, :].astype(jnp.float32), m2, brow2)
    part2 = sc2(tx2_ext, src3, dst3, att2t2)

    out = _tc_final(part2[0], part2[1], bias2[None, :].astype(jnp.float32))
    return out[0:N]
